# Initial kernel scaffold; baseline (speedup 1.0000x reference)
import jax
import jax.numpy as jnp
from jax.experimental import pallas as pl


def kernel(x, edge_index, W1, b1, W2, b2):
    raise NotImplementedError("write your pallas kernel here")


# speedup vs baseline: 20.1077x; 20.1077x over previous
import functools

import jax
import jax.numpy as jnp
from jax import lax
from jax.experimental import pallas as pl
from jax.experimental.pallas import tpu as pltpu
from jax.experimental.pallas import tpu_sc as plsc

N = 10000
E = 320000
D = 128

N_PAD = 10240
NC = 2
NS = 16
WIN = 128
STEPS = 80
E_PAD = NC * NS * STEPS * WIN
ROWS_PER_SC = N_PAD // NS

_MESH = plsc.VectorSubcoreMesh(core_axis_name="c", subcore_axis_name="s")


def _deg_body(dst_hbm, zvec_hbm, out_hbm, dstv, onesv, deg_sh, sem):
    cid = lax.axis_index("c")
    sid = lax.axis_index("s")
    base = (cid * NS + sid) * STEPS

    pltpu.sync_copy(dst_hbm.at[pl.ds(base, STEPS)], dstv)
    for j in range(WIN // 16):
        onesv[pl.ds(j * 16, 16)] = jnp.ones((16,), jnp.float32)

    pltpu.sync_copy(zvec_hbm, deg_sh.at[pl.ds(sid * ROWS_PER_SC, ROWS_PER_SC)])
    plsc.subcore_barrier()

    def step(t, _):
        pltpu.sync_copy(onesv, deg_sh.at[dstv.at[t]], add=True)
        return 0

    lax.fori_loop(0, STEPS, step, 0)
    plsc.subcore_barrier()

    pltpu.sync_copy(
        deg_sh.at[pl.ds(sid * ROWS_PER_SC, ROWS_PER_SC)],
        out_hbm.at[cid, pl.ds(sid * ROWS_PER_SC, ROWS_PER_SC)],
    )


@functools.partial(jax.jit, donate_argnums=())
def _deg_kernel(dst2d, zvec):
    return pl.kernel(
        _deg_body,
        out_type=jax.ShapeDtypeStruct((NC, N_PAD), jnp.float32),
        mesh=_MESH,
        scratch_types=[
            pltpu.VMEM((STEPS, WIN), jnp.int32),
            pltpu.VMEM((WIN,), jnp.float32),
            pltpu.VMEM_SHARED((N_PAD,), jnp.float32),
            pltpu.SemaphoreType.DMA,
        ],
    )(dst2d, zvec)


def _agg_body(h_hbm, src_hbm, dst_hbm, zrows_hbm, out_hbm,
              srcv, dstv, rowsv, acc_sh, sem):
    cid = lax.axis_index("c")
    sid = lax.axis_index("s")
    base = (cid * NS + sid) * STEPS

    pltpu.sync_copy(src_hbm.at[pl.ds(base, STEPS)], srcv)
    pltpu.sync_copy(dst_hbm.at[pl.ds(base, STEPS)], dstv)

    for k in range(ROWS_PER_SC // 80):
        pltpu.sync_copy(
            zrows_hbm, acc_sh.at[pl.ds(sid * ROWS_PER_SC + k * 80, 80)]
        )
    plsc.subcore_barrier()

    def step(t, _):
        pltpu.async_copy(h_hbm.at[srcv.at[t]], rowsv, sem).wait()
        pltpu.sync_copy(rowsv, acc_sh.at[dstv.at[t]], add=True)
        return 0

    lax.fori_loop(0, STEPS, step, 0)
    plsc.subcore_barrier()

    pltpu.sync_copy(
        acc_sh.at[pl.ds(sid * ROWS_PER_SC, ROWS_PER_SC)],
        out_hbm.at[cid, pl.ds(sid * ROWS_PER_SC, ROWS_PER_SC)],
    )


@jax.jit
def _agg_kernel(h, src2d, dst2d, zrows):
    return pl.kernel(
        _agg_body,
        out_type=jax.ShapeDtypeStruct((NC, N_PAD, D), jnp.float32),
        mesh=_MESH,
        scratch_types=[
            pltpu.VMEM((STEPS, WIN), jnp.int32),
            pltpu.VMEM((STEPS, WIN), jnp.int32),
            pltpu.VMEM((WIN, D), jnp.float32),
            pltpu.VMEM_SHARED((N_PAD, D), jnp.float32),
            pltpu.SemaphoreType.DMA,
        ],
    )(h, src2d, dst2d, zrows)


_R = 512
_G = N_PAD // _R


def _tc_b_body(x_ref, w_ref, deg_ref, h_ref, dis_ref):
    deg = deg_ref[0, :] + deg_ref[1, :] + 1.0
    dis = lax.rsqrt(deg)[:, None]
    h = lax.dot_general(
        x_ref[...], w_ref[...], (((1,), (1,)), ((), ())),
        preferred_element_type=jnp.float32,
    )
    h_ref[...] = h * dis
    dis_ref[...] = dis


@jax.jit
def _tc_b(xp, W1, deg_parts):
    return pl.pallas_call(
        _tc_b_body,
        grid=(_G,),
        in_specs=[
            pl.BlockSpec((_R, D), lambda i: (i, 0)),
            pl.BlockSpec((D, D), lambda i: (0, 0)),
            pl.BlockSpec((NC, _R), lambda i: (0, i)),
        ],
        out_specs=[
            pl.BlockSpec((_R, D), lambda i: (i, 0)),
            pl.BlockSpec((_R, 1), lambda i: (i, 0)),
        ],
        out_shape=[
            jax.ShapeDtypeStruct((N_PAD, D), jnp.float32),
            jax.ShapeDtypeStruct((N_PAD, 1), jnp.float32),
        ],
    )(xp, W1, deg_parts)


def _tc_c_body(p0_ref, p1_ref, h_ref, dis_ref, w_ref, b_ref, out_ref):
    dis = dis_ref[...]
    z = dis * (p0_ref[...] + p1_ref[...] + h_ref[...]) + b_ref[...]
    h = jnp.maximum(z, 0.0)
    out_ref[...] = dis * lax.dot_general(
        h, w_ref[...], (((1,), (1,)), ((), ())),
        preferred_element_type=jnp.float32,
    )


@jax.jit
def _tc_c(p0, p1, h1p, dis, W2, b1):
    return pl.pallas_call(
        _tc_c_body,
        grid=(_G,),
        in_specs=[
            pl.BlockSpec((_R, D), lambda i: (i, 0)),
            pl.BlockSpec((_R, D), lambda i: (i, 0)),
            pl.BlockSpec((_R, D), lambda i: (i, 0)),
            pl.BlockSpec((_R, 1), lambda i: (i, 0)),
            pl.BlockSpec((D, D), lambda i: (0, 0)),
            pl.BlockSpec((1, D), lambda i: (0, 0)),
        ],
        out_specs=pl.BlockSpec((_R, D), lambda i: (i, 0)),
        out_shape=jax.ShapeDtypeStruct((N_PAD, D), jnp.float32),
    )(p0, p1, h1p, dis, W2, b1)


def _tc_d_body(p0_ref, p1_ref, h_ref, dis_ref, b_ref, out_ref):
    out_ref[...] = (
        dis_ref[...] * (p0_ref[...] + p1_ref[...] + h_ref[...]) + b_ref[...]
    )


@jax.jit
def _tc_d(p0, p1, h2p, dis, b2):
    return pl.pallas_call(
        _tc_d_body,
        grid=(_G,),
        in_specs=[
            pl.BlockSpec((_R, D), lambda i: (i, 0)),
            pl.BlockSpec((_R, D), lambda i: (i, 0)),
            pl.BlockSpec((_R, D), lambda i: (i, 0)),
            pl.BlockSpec((_R, 1), lambda i: (i, 0)),
            pl.BlockSpec((1, D), lambda i: (0, 0)),
        ],
        out_specs=pl.BlockSpec((_R, D), lambda i: (i, 0)),
        out_shape=jax.ShapeDtypeStruct((N_PAD, D), jnp.float32),
    )(p0, p1, h2p, dis, b2)


def kernel(x, edge_index, W1, b1, W2, b2):
    src = edge_index[0].astype(jnp.int32)
    dst = edge_index[1].astype(jnp.int32)

    pad_n = E_PAD - E
    pad_ar = jnp.arange(pad_n, dtype=jnp.int32)
    src_p = jnp.concatenate([src, pad_ar % N])
    dst_p = jnp.concatenate([dst, N + pad_ar % (N_PAD - N)])
    src2d = src_p.reshape(E_PAD // WIN, WIN)
    dst2d = dst_p.reshape(E_PAD // WIN, WIN)

    xp = jnp.zeros((N_PAD, D), jnp.float32).at[:N].set(x)
    zvec = jnp.zeros((ROWS_PER_SC,), jnp.float32)
    zrows = jnp.zeros((80, D), jnp.float32)

    deg_parts = _deg_kernel(dst2d, zvec)
    h1p, dis = _tc_b(xp, W1, deg_parts)
    parts1 = _agg_kernel(h1p, src2d, dst2d, zrows)
    h2p = _tc_c(parts1[0], parts1[1], h1p, dis, W2,
                b1.reshape(1, D))
    parts2 = _agg_kernel(h2p, src2d, dst2d, zrows)
    outp = _tc_d(parts2[0], parts2[1], h2p, dis, b2.reshape(1, D))
    return outp[:N]


# speedup vs baseline: 24.9131x; 1.2390x over previous
import functools

import jax
import jax.numpy as jnp
from jax import lax
from jax.experimental import pallas as pl
from jax.experimental.pallas import tpu as pltpu
from jax.experimental.pallas import tpu_sc as plsc

N = 10000
E = 320000
D = 128

N_PAD = 10240
NC = 2
NS = 16
WIN = 128
STEPS = 80
E_PAD = NC * NS * STEPS * WIN
ROWS_PER_SC = N_PAD // NS

_MESH = plsc.VectorSubcoreMesh(core_axis_name="c", subcore_axis_name="s")


def _win_slice(hbm, ebase, t):
    start = pl.multiple_of(ebase + t * WIN, WIN)
    return hbm.at[pl.ds(start, WIN)]


def _deg_body(dst_hbm, zvec_hbm, out_hbm, di0, di1, onesv, is0, is1, deg_sh):
    cid = lax.axis_index("c")
    sid = lax.axis_index("s")
    ebase = (cid * NS + sid) * STEPS * WIN
    dib = (di0, di1)
    isems = (is0, is1)

    for j in range(WIN // 16):
        onesv[pl.ds(j * 16, 16)] = jnp.ones((16,), jnp.float32)

    for t in range(2):
        pltpu.async_copy(_win_slice(dst_hbm, ebase, t), dib[t], isems[t])
    pltpu.sync_copy(zvec_hbm, deg_sh.at[pl.ds(sid * ROWS_PER_SC, ROWS_PER_SC)])
    plsc.subcore_barrier()

    def pair(i, _):
        for b in range(2):
            t = 2 * i + b
            pltpu.make_async_copy(
                _win_slice(dst_hbm, ebase, t), dib[b], isems[b]
            ).wait()
            pltpu.sync_copy(onesv, deg_sh.at[dib[b]], add=True)

            @pl.when(t + 2 < STEPS)
            def _():
                pltpu.async_copy(
                    _win_slice(dst_hbm, ebase, t + 2), dib[b], isems[b]
                )

        return 0

    lax.fori_loop(0, STEPS // 2, pair, 0)
    plsc.subcore_barrier()

    pltpu.sync_copy(
        deg_sh.at[pl.ds(sid * ROWS_PER_SC, ROWS_PER_SC)],
        out_hbm.at[cid, pl.ds(sid * ROWS_PER_SC, ROWS_PER_SC)],
    )


@jax.jit
def _deg_kernel(dst1d, zvec):
    return pl.kernel(
        _deg_body,
        out_type=jax.ShapeDtypeStruct((NC, N_PAD), jnp.float32),
        mesh=_MESH,
        scratch_types=[
            pltpu.VMEM((WIN,), jnp.int32),
            pltpu.VMEM((WIN,), jnp.int32),
            pltpu.VMEM((WIN,), jnp.float32),
            pltpu.SemaphoreType.DMA,
            pltpu.SemaphoreType.DMA,
            pltpu.VMEM_SHARED((N_PAD,), jnp.float32),
        ],
    )(dst1d, zvec)


def _agg_body(h_hbm, src_hbm, dst_hbm, zrows_hbm, out_hbm,
              si0, si1, di0, di1, rows0, rows1,
              is0, is1, g0, g1, ssem, acc_sh):
    cid = lax.axis_index("c")
    sid = lax.axis_index("s")
    ebase = (cid * NS + sid) * STEPS * WIN
    sib = (si0, si1)
    dib = (di0, di1)
    rbufs = (rows0, rows1)
    isems = (is0, is1)
    gsems = (g0, g1)

    def idx_fire(t, b):
        pltpu.async_copy(_win_slice(src_hbm, ebase, t), sib[b], isems[b])
        pltpu.async_copy(_win_slice(dst_hbm, ebase, t), dib[b], isems[b])

    def idx_wait(t, b):
        pltpu.make_async_copy(_win_slice(src_hbm, ebase, t), sib[b], isems[b]).wait()
        pltpu.make_async_copy(_win_slice(dst_hbm, ebase, t), dib[b], isems[b]).wait()

    idx_fire(0, 0)
    idx_fire(1, 1)
    for k in range(ROWS_PER_SC // 80):
        pltpu.sync_copy(
            zrows_hbm, acc_sh.at[pl.ds(sid * ROWS_PER_SC + k * 80, 80)]
        )
    idx_wait(0, 0)
    pltpu.async_copy(h_hbm.at[sib[0]], rbufs[0], gsems[0])
    plsc.subcore_barrier()

    def pair(i, _):
        for b in range(2):
            t = 2 * i + b
            nb = 1 - b

            @pl.when(t + 1 < STEPS)
            def _():
                idx_wait(t + 1, nb)
                pltpu.async_copy(h_hbm.at[sib[nb]], rbufs[nb], gsems[nb])

            pltpu.make_async_copy(h_hbm.at[sib[b]], rbufs[b], gsems[b]).wait()
            sdesc = pltpu.async_copy(
                rbufs[b], acc_sh.at[dib[b]], ssem, add=True
            )
            sdesc.wait()

            @pl.when(t + 2 < STEPS)
            def _():
                idx_fire(t + 2, b)

        return 0

    lax.fori_loop(0, STEPS // 2, pair, 0)
    plsc.subcore_barrier()

    pltpu.sync_copy(
        acc_sh.at[pl.ds(sid * ROWS_PER_SC, ROWS_PER_SC)],
        out_hbm.at[cid, pl.ds(sid * ROWS_PER_SC, ROWS_PER_SC)],
    )


@jax.jit
def _agg_kernel(h, src1d, dst1d, zrows):
    return pl.kernel(
        _agg_body,
        out_type=jax.ShapeDtypeStruct((NC, N_PAD, D), jnp.float32),
        mesh=_MESH,
        scratch_types=[
            pltpu.VMEM((WIN,), jnp.int32),
            pltpu.VMEM((WIN,), jnp.int32),
            pltpu.VMEM((WIN,), jnp.int32),
            pltpu.VMEM((WIN,), jnp.int32),
            pltpu.VMEM((WIN, D), jnp.float32),
            pltpu.VMEM((WIN, D), jnp.float32),
            pltpu.SemaphoreType.DMA,
            pltpu.SemaphoreType.DMA,
            pltpu.SemaphoreType.DMA,
            pltpu.SemaphoreType.DMA,
            pltpu.SemaphoreType.DMA,
            pltpu.VMEM_SHARED((N_PAD, D), jnp.float32),
        ],
    )(h, src1d, dst1d, zrows)


_R = 512
_G = N_PAD // _R


def _tc_b_body(x_ref, w_ref, deg_ref, h_ref, dis_ref):
    deg = deg_ref[0, :] + deg_ref[1, :] + 1.0
    dis = lax.rsqrt(deg)[:, None]
    h = lax.dot_general(
        x_ref[...], w_ref[...], (((1,), (1,)), ((), ())),
        preferred_element_type=jnp.float32,
    )
    h_ref[...] = h * dis
    dis_ref[...] = dis


@jax.jit
def _tc_b(xp, W1, deg_parts):
    return pl.pallas_call(
        _tc_b_body,
        grid=(_G,),
        in_specs=[
            pl.BlockSpec((_R, D), lambda i: (i, 0)),
            pl.BlockSpec((D, D), lambda i: (0, 0)),
            pl.BlockSpec((NC, _R), lambda i: (0, i)),
        ],
        out_specs=[
            pl.BlockSpec((_R, D), lambda i: (i, 0)),
            pl.BlockSpec((_R, 1), lambda i: (i, 0)),
        ],
        out_shape=[
            jax.ShapeDtypeStruct((N_PAD, D), jnp.float32),
            jax.ShapeDtypeStruct((N_PAD, 1), jnp.float32),
        ],
    )(xp, W1, deg_parts)


def _tc_c_body(p0_ref, p1_ref, h_ref, dis_ref, w_ref, b_ref, out_ref):
    dis = dis_ref[...]
    z = dis * (p0_ref[...] + p1_ref[...] + h_ref[...]) + b_ref[...]
    h = jnp.maximum(z, 0.0)
    out_ref[...] = dis * lax.dot_general(
        h, w_ref[...], (((1,), (1,)), ((), ())),
        preferred_element_type=jnp.float32,
    )


@jax.jit
def _tc_c(p0, p1, h1p, dis, W2, b1):
    return pl.pallas_call(
        _tc_c_body,
        grid=(_G,),
        in_specs=[
            pl.BlockSpec((_R, D), lambda i: (i, 0)),
            pl.BlockSpec((_R, D), lambda i: (i, 0)),
            pl.BlockSpec((_R, D), lambda i: (i, 0)),
            pl.BlockSpec((_R, 1), lambda i: (i, 0)),
            pl.BlockSpec((D, D), lambda i: (0, 0)),
            pl.BlockSpec((1, D), lambda i: (0, 0)),
        ],
        out_specs=pl.BlockSpec((_R, D), lambda i: (i, 0)),
        out_shape=jax.ShapeDtypeStruct((N_PAD, D), jnp.float32),
    )(p0, p1, h1p, dis, W2, b1)


def _tc_d_body(p0_ref, p1_ref, h_ref, dis_ref, b_ref, out_ref):
    out_ref[...] = (
        dis_ref[...] * (p0_ref[...] + p1_ref[...] + h_ref[...]) + b_ref[...]
    )


@jax.jit
def _tc_d(p0, p1, h2p, dis, b2):
    return pl.pallas_call(
        _tc_d_body,
        grid=(_G,),
        in_specs=[
            pl.BlockSpec((_R, D), lambda i: (i, 0)),
            pl.BlockSpec((_R, D), lambda i: (i, 0)),
            pl.BlockSpec((_R, D), lambda i: (i, 0)),
            pl.BlockSpec((_R, 1), lambda i: (i, 0)),
            pl.BlockSpec((1, D), lambda i: (0, 0)),
        ],
        out_specs=pl.BlockSpec((_R, D), lambda i: (i, 0)),
        out_shape=jax.ShapeDtypeStruct((N_PAD, D), jnp.float32),
    )(p0, p1, h2p, dis, b2)


def kernel(x, edge_index, W1, b1, W2, b2):
    src = edge_index[0].astype(jnp.int32)
    dst = edge_index[1].astype(jnp.int32)

    pad_n = E_PAD - E
    pad_ar = jnp.arange(pad_n, dtype=jnp.int32)
    src_p = jnp.concatenate([src, pad_ar % N])
    dst_p = jnp.concatenate([dst, N + pad_ar % (N_PAD - N)])

    zvec = jnp.zeros((ROWS_PER_SC,), jnp.float32)
    zrows = jnp.zeros((80, D), jnp.float32)

    deg_parts = _deg_kernel(dst_p, zvec)
    h1p, dis = _tc_b(x, W1, deg_parts)
    parts1 = _agg_kernel(h1p, src_p, dst_p, zrows)
    h2p = _tc_c(parts1[0], parts1[1], h1p, dis, W2,
                b1.reshape(1, D))
    parts2 = _agg_kernel(h2p, src_p, dst_p, zrows)
    outp = _tc_d(parts2[0], parts2[1], h2p, dis, b2.reshape(1, D))
    return outp[:N]


# speedup vs baseline: 25.7529x; 1.0337x over previous
import functools

import jax
import jax.numpy as jnp
from jax import lax
from jax.experimental import pallas as pl
from jax.experimental.pallas import tpu as pltpu
from jax.experimental.pallas import tpu_sc as plsc

N = 10000
E = 320000
D = 128

N_PAD = 10240
NC = 2
NS = 16
WIN = 128
STEPS = 80
E_PAD = NC * NS * STEPS * WIN
ROWS_PER_SC = N_PAD // NS

_MESH = plsc.VectorSubcoreMesh(core_axis_name="c", subcore_axis_name="s")


def _win_slice(hbm, ebase, t):
    start = pl.multiple_of(ebase + t * WIN, WIN)
    return hbm.at[pl.ds(start, WIN)]


def _deg_body(dst2d_hbm, zvec_hbm, out_hbm, dstv, onesv, sem, deg_sh):
    cid = lax.axis_index("c")
    sid = lax.axis_index("s")
    base = (cid * NS + sid) * STEPS

    pltpu.sync_copy(dst2d_hbm.at[pl.ds(base, STEPS)], dstv)
    for j in range(WIN // 16):
        onesv[pl.ds(j * 16, 16)] = jnp.ones((16,), jnp.float32)

    pltpu.sync_copy(zvec_hbm, deg_sh.at[pl.ds(sid * ROWS_PER_SC, ROWS_PER_SC)])
    plsc.subcore_barrier()

    def step(t, _):
        pltpu.sync_copy(onesv, deg_sh.at[dstv.at[t]], add=True)
        return 0

    lax.fori_loop(0, STEPS, step, 0)
    plsc.subcore_barrier()

    pltpu.sync_copy(
        deg_sh.at[pl.ds(sid * ROWS_PER_SC, ROWS_PER_SC)],
        out_hbm.at[cid, pl.ds(sid * ROWS_PER_SC, ROWS_PER_SC)],
    )


@jax.jit
def _deg_kernel(dst2d, zvec):
    return pl.kernel(
        _deg_body,
        out_type=jax.ShapeDtypeStruct((NC, N_PAD), jnp.float32),
        mesh=_MESH,
        scratch_types=[
            pltpu.VMEM((STEPS, WIN), jnp.int32),
            pltpu.VMEM((WIN,), jnp.float32),
            pltpu.SemaphoreType.DMA,
            pltpu.VMEM_SHARED((N_PAD,), jnp.float32),
        ],
    )(dst2d, zvec)


def _agg_body(h_hbm, src_hbm, dst_hbm, zrows_hbm, out_hbm,
              si0, si1, si2, si3, di0, di1, di2, di3, rows0, rows1,
              is0, is1, is2, is3, g0, g1, ss0, ss1, acc_sh):
    cid = lax.axis_index("c")
    sid = lax.axis_index("s")
    ebase = (cid * NS + sid) * STEPS * WIN
    sib = (si0, si1, si2, si3)
    dib = (di0, di1, di2, di3)
    rbufs = (rows0, rows1)
    isems = (is0, is1, is2, is3)
    gsems = (g0, g1)
    ssems = (ss0, ss1)

    def idx_fire(t, q):
        pltpu.async_copy(_win_slice(src_hbm, ebase, t), sib[q], isems[q])
        pltpu.async_copy(_win_slice(dst_hbm, ebase, t), dib[q], isems[q])

    def idx_wait(t, q):
        pltpu.make_async_copy(_win_slice(src_hbm, ebase, t), sib[q], isems[q]).wait()
        pltpu.make_async_copy(_win_slice(dst_hbm, ebase, t), dib[q], isems[q]).wait()

    def scat_wait(b, q):
        pltpu.make_async_copy(rbufs[b], acc_sh.at[dib[q]], ssems[b]).wait()

    idx_fire(0, 0)
    idx_fire(1, 1)
    for k in range(ROWS_PER_SC // 80):
        pltpu.sync_copy(
            zrows_hbm, acc_sh.at[pl.ds(sid * ROWS_PER_SC + k * 80, 80)]
        )
    idx_wait(0, 0)
    pltpu.async_copy(h_hbm.at[sib[0]], rbufs[0], gsems[0])
    plsc.subcore_barrier()

    def quad(i, _):
        for j in range(4):
            t = 4 * i + j
            b = j % 2
            nb = 1 - b
            q = j

            @pl.when(t + 1 < STEPS)
            def _():
                idx_wait(t + 1, (j + 1) % 4)
                pltpu.async_copy(h_hbm.at[sib[(j + 1) % 4]], rbufs[nb], gsems[nb])

            pltpu.make_async_copy(h_hbm.at[sib[q]], rbufs[b], gsems[b]).wait()
            pltpu.async_copy(rbufs[b], acc_sh.at[dib[q]], ssems[b], add=True)
            scat_wait(b, q)

            @pl.when(t + 2 < STEPS)
            def _():
                idx_fire(t + 2, (j + 2) % 4)

        return 0

    lax.fori_loop(0, STEPS // 4, quad, 0)
    plsc.subcore_barrier()

    pltpu.sync_copy(
        acc_sh.at[pl.ds(sid * ROWS_PER_SC, ROWS_PER_SC)],
        out_hbm.at[cid, pl.ds(sid * ROWS_PER_SC, ROWS_PER_SC)],
    )


@jax.jit
def _agg_kernel(h, src1d, dst1d, zrows):
    return pl.kernel(
        _agg_body,
        out_type=jax.ShapeDtypeStruct((NC, N_PAD, D), jnp.float32),
        mesh=_MESH,
        scratch_types=(
            [pltpu.VMEM((WIN,), jnp.int32)] * 8
            + [pltpu.VMEM((WIN, D), jnp.float32)] * 2
            + [pltpu.SemaphoreType.DMA] * 8
            + [pltpu.VMEM_SHARED((N_PAD, D), jnp.float32)]
        ),
    )(h, src1d, dst1d, zrows)


_R = 512
_G = N_PAD // _R


def _tc_b_body(x_ref, w_ref, deg_ref, h_ref, dis_ref):
    deg = deg_ref[0, :] + deg_ref[1, :] + 1.0
    dis = lax.rsqrt(deg)[:, None]
    h = lax.dot_general(
        x_ref[...], w_ref[...], (((1,), (1,)), ((), ())),
        preferred_element_type=jnp.float32,
    )
    h_ref[...] = h * dis
    dis_ref[...] = dis


@jax.jit
def _tc_b(xp, W1, deg_parts):
    return pl.pallas_call(
        _tc_b_body,
        grid=(_G,),
        in_specs=[
            pl.BlockSpec((_R, D), lambda i: (i, 0)),
            pl.BlockSpec((D, D), lambda i: (0, 0)),
            pl.BlockSpec((NC, _R), lambda i: (0, i)),
        ],
        out_specs=[
            pl.BlockSpec((_R, D), lambda i: (i, 0)),
            pl.BlockSpec((_R, 1), lambda i: (i, 0)),
        ],
        out_shape=[
            jax.ShapeDtypeStruct((N_PAD, D), jnp.float32),
            jax.ShapeDtypeStruct((N_PAD, 1), jnp.float32),
        ],
    )(xp, W1, deg_parts)


def _tc_c_body(p0_ref, p1_ref, h_ref, dis_ref, w_ref, b_ref, out_ref):
    dis = dis_ref[...]
    z = dis * (p0_ref[...] + p1_ref[...] + h_ref[...]) + b_ref[...]
    h = jnp.maximum(z, 0.0)
    out_ref[...] = dis * lax.dot_general(
        h, w_ref[...], (((1,), (1,)), ((), ())),
        preferred_element_type=jnp.float32,
    )


@jax.jit
def _tc_c(p0, p1, h1p, dis, W2, b1):
    return pl.pallas_call(
        _tc_c_body,
        grid=(_G,),
        in_specs=[
            pl.BlockSpec((_R, D), lambda i: (i, 0)),
            pl.BlockSpec((_R, D), lambda i: (i, 0)),
            pl.BlockSpec((_R, D), lambda i: (i, 0)),
            pl.BlockSpec((_R, 1), lambda i: (i, 0)),
            pl.BlockSpec((D, D), lambda i: (0, 0)),
            pl.BlockSpec((1, D), lambda i: (0, 0)),
        ],
        out_specs=pl.BlockSpec((_R, D), lambda i: (i, 0)),
        out_shape=jax.ShapeDtypeStruct((N_PAD, D), jnp.float32),
    )(p0, p1, h1p, dis, W2, b1)


def _tc_d_body(p0_ref, p1_ref, h_ref, dis_ref, b_ref, out_ref):
    out_ref[...] = (
        dis_ref[...] * (p0_ref[...] + p1_ref[...] + h_ref[...]) + b_ref[...]
    )


@jax.jit
def _tc_d(p0, p1, h2p, dis, b2):
    return pl.pallas_call(
        _tc_d_body,
        grid=(_G,),
        in_specs=[
            pl.BlockSpec((_R, D), lambda i: (i, 0)),
            pl.BlockSpec((_R, D), lambda i: (i, 0)),
            pl.BlockSpec((_R, D), lambda i: (i, 0)),
            pl.BlockSpec((_R, 1), lambda i: (i, 0)),
            pl.BlockSpec((1, D), lambda i: (0, 0)),
        ],
        out_specs=pl.BlockSpec((_R, D), lambda i: (i, 0)),
        out_shape=jax.ShapeDtypeStruct((N_PAD, D), jnp.float32),
    )(p0, p1, h2p, dis, b2)


def kernel(x, edge_index, W1, b1, W2, b2):
    src = edge_index[0].astype(jnp.int32)
    dst = edge_index[1].astype(jnp.int32)

    pad_n = E_PAD - E
    pad_ar = jnp.arange(pad_n, dtype=jnp.int32)
    src_p = jnp.concatenate([src, pad_ar % N])
    dst_p = jnp.concatenate([dst, N + pad_ar % (N_PAD - N)])

    zvec = jnp.zeros((ROWS_PER_SC,), jnp.float32)
    zrows = jnp.zeros((80, D), jnp.float32)

    deg_parts = _deg_kernel(dst_p.reshape(-1, WIN), zvec)
    h1p, dis = _tc_b(x, W1, deg_parts)
    parts1 = _agg_kernel(h1p, src_p, dst_p, zrows)
    h2p = _tc_c(parts1[0], parts1[1], h1p, dis, W2,
                b1.reshape(1, D))
    parts2 = _agg_kernel(h2p, src_p, dst_p, zrows)
    outp = _tc_d(parts2[0], parts2[1], h2p, dis, b2.reshape(1, D))
    return outp[:N]


# speedup vs baseline: 28.0821x; 1.0904x over previous
import functools

import jax
import jax.numpy as jnp
from jax import lax
from jax.experimental import pallas as pl
from jax.experimental.pallas import tpu as pltpu
from jax.experimental.pallas import tpu_sc as plsc

N = 10000
E = 320000
D = 128

N_PAD = 10240
NC = 2
NS = 16
WIN = 128
STEPS = 80
E_PAD = NC * NS * STEPS * WIN
ROWS_PER_SC = N_PAD // NS

_MESH = plsc.VectorSubcoreMesh(core_axis_name="c", subcore_axis_name="s")


def _win_slice(hbm, ebase, t):
    start = pl.multiple_of(ebase + t * WIN, WIN)
    return hbm.at[pl.ds(start, WIN)]


def _deg_body(dst2d_hbm, zvec_hbm, out_hbm, dstv, onesv, sem, deg_sh):
    cid = lax.axis_index("c")
    sid = lax.axis_index("s")
    base = (cid * NS + sid) * STEPS

    pltpu.sync_copy(dst2d_hbm.at[pl.ds(base, STEPS)], dstv)
    for j in range(WIN // 16):
        onesv[pl.ds(j * 16, 16)] = jnp.ones((16,), jnp.float32)

    pltpu.sync_copy(zvec_hbm, deg_sh.at[pl.ds(sid * ROWS_PER_SC, ROWS_PER_SC)])
    plsc.subcore_barrier()

    def step(t, _):
        pltpu.sync_copy(onesv, deg_sh.at[dstv.at[t]], add=True)
        return 0

    lax.fori_loop(0, STEPS, step, 0)
    plsc.subcore_barrier()

    pltpu.sync_copy(
        deg_sh.at[pl.ds(sid * ROWS_PER_SC, ROWS_PER_SC)],
        out_hbm.at[cid, pl.ds(sid * ROWS_PER_SC, ROWS_PER_SC)],
    )


@jax.jit
def _deg_kernel(dst2d, zvec):
    return pl.kernel(
        _deg_body,
        out_type=jax.ShapeDtypeStruct((NC, N_PAD), jnp.float32),
        mesh=_MESH,
        scratch_types=[
            pltpu.VMEM((STEPS, WIN), jnp.int32),
            pltpu.VMEM((WIN,), jnp.float32),
            pltpu.SemaphoreType.DMA,
            pltpu.VMEM_SHARED((N_PAD,), jnp.float32),
        ],
    )(dst2d, zvec)


def _agg_body(h_hbm, src_hbm, dst_hbm, zrows_hbm, out_hbm,
              si0, si1, si2, si3, di0, di1, di2, di3, rows0, rows1,
              is0, is1, is2, is3, g0, g1, ss0, ss1, acc_sh):
    cid = lax.axis_index("c")
    sid = lax.axis_index("s")
    ebase = (cid * NS + sid) * STEPS * WIN
    sib = (si0, si1, si2, si3)
    dib = (di0, di1, di2, di3)
    rbufs = (rows0, rows1)
    isems = (is0, is1, is2, is3)
    gsems = (g0, g1)
    ssems = (ss0, ss1)

    def idx_fire(t, q):
        pltpu.async_copy(_win_slice(src_hbm, ebase, t), sib[q], isems[q])
        pltpu.async_copy(_win_slice(dst_hbm, ebase, t), dib[q], isems[q])

    def idx_wait(t, q):
        pltpu.make_async_copy(_win_slice(src_hbm, ebase, t), sib[q], isems[q]).wait()
        pltpu.make_async_copy(_win_slice(dst_hbm, ebase, t), dib[q], isems[q]).wait()

    def scat_wait(b, q):
        pltpu.make_async_copy(rbufs[b], acc_sh.at[dib[q]], ssems[b]).wait()

    idx_fire(0, 0)
    idx_fire(1, 1)
    for k in range(ROWS_PER_SC // 80):
        pltpu.sync_copy(
            zrows_hbm, acc_sh.at[pl.ds(sid * ROWS_PER_SC + k * 80, 80)]
        )
    idx_wait(0, 0)
    pltpu.async_copy(h_hbm.at[sib[0]], rbufs[0], gsems[0])
    plsc.subcore_barrier()

    def quad(i, _):
        for j in range(4):
            t = 4 * i + j
            b = j % 2
            nb = 1 - b
            q = j

            @pl.when(t + 1 < STEPS)
            def _():
                idx_wait(t + 1, (j + 1) % 4)
                pltpu.async_copy(h_hbm.at[sib[(j + 1) % 4]], rbufs[nb], gsems[nb])

            pltpu.make_async_copy(h_hbm.at[sib[q]], rbufs[b], gsems[b]).wait()
            pltpu.async_copy(rbufs[b], acc_sh.at[dib[q]], ssems[b], add=True)
            scat_wait(b, q)

            @pl.when(t + 2 < STEPS)
            def _():
                idx_fire(t + 2, (j + 2) % 4)

        return 0

    lax.fori_loop(0, STEPS // 4, quad, 0)
    plsc.subcore_barrier()

    pltpu.sync_copy(
        acc_sh.at[pl.ds(sid * ROWS_PER_SC, ROWS_PER_SC)],
        out_hbm.at[cid, pl.ds(sid * ROWS_PER_SC, ROWS_PER_SC)],
    )


@jax.jit
def _agg_kernel(h, src1d, dst1d, zrows):
    return pl.kernel(
        _agg_body,
        out_type=jax.ShapeDtypeStruct((NC, N_PAD, D), jnp.float32),
        mesh=_MESH,
        scratch_types=(
            [pltpu.VMEM((WIN,), jnp.int32)] * 8
            + [pltpu.VMEM((WIN, D), jnp.float32)] * 2
            + [pltpu.SemaphoreType.DMA] * 8
            + [pltpu.VMEM_SHARED((N_PAD, D), jnp.float32)]
        ),
    )(h, src1d, dst1d, zrows)


_R = 1024
_G = N_PAD // _R


def _tc_b_body(x_ref, w_ref, deg_ref, h_ref, dis_ref):
    deg = deg_ref[0, :] + deg_ref[1, :] + 1.0
    dis = lax.rsqrt(deg)[:, None]
    h = lax.dot_general(
        x_ref[...], w_ref[...], (((1,), (1,)), ((), ())),
        preferred_element_type=jnp.float32,
    )
    h_ref[...] = h * dis
    dis_ref[...] = dis


@jax.jit
def _tc_b(xp, W1, deg_parts):
    return pl.pallas_call(
        _tc_b_body,
        grid=(_G,),
        in_specs=[
            pl.BlockSpec((_R, D), lambda i: (i, 0)),
            pl.BlockSpec((D, D), lambda i: (0, 0)),
            pl.BlockSpec((NC, _R), lambda i: (0, i)),
        ],
        out_specs=[
            pl.BlockSpec((_R, D), lambda i: (i, 0)),
            pl.BlockSpec((_R, 1), lambda i: (i, 0)),
        ],
        out_shape=[
            jax.ShapeDtypeStruct((N_PAD, D), jnp.float32),
            jax.ShapeDtypeStruct((N_PAD, 1), jnp.float32),
        ],
    )(xp, W1, deg_parts)


def _tc_c_body(parts_ref, h_ref, dis_ref, w_ref, b_ref, out_ref):
    dis = dis_ref[...]
    z = dis * (parts_ref[0] + parts_ref[1] + h_ref[...]) + b_ref[...]
    h = jnp.maximum(z, 0.0)
    out_ref[...] = dis * lax.dot_general(
        h, w_ref[...], (((1,), (1,)), ((), ())),
        preferred_element_type=jnp.float32,
    )


@jax.jit
def _tc_c(parts, h1p, dis, W2, b1):
    return pl.pallas_call(
        _tc_c_body,
        grid=(_G,),
        in_specs=[
            pl.BlockSpec((NC, _R, D), lambda i: (0, i, 0)),
            pl.BlockSpec((_R, D), lambda i: (i, 0)),
            pl.BlockSpec((_R, 1), lambda i: (i, 0)),
            pl.BlockSpec((D, D), lambda i: (0, 0)),
            pl.BlockSpec((1, D), lambda i: (0, 0)),
        ],
        out_specs=pl.BlockSpec((_R, D), lambda i: (i, 0)),
        out_shape=jax.ShapeDtypeStruct((N_PAD, D), jnp.float32),
    )(parts, h1p, dis, W2, b1)


def _tc_d_body(parts_ref, h_ref, dis_ref, b_ref, out_ref):
    out_ref[...] = (
        dis_ref[...] * (parts_ref[0] + parts_ref[1] + h_ref[...]) + b_ref[...]
    )


@jax.jit
def _tc_d(parts, h2p, dis, b2):
    return pl.pallas_call(
        _tc_d_body,
        grid=(_G,),
        in_specs=[
            pl.BlockSpec((NC, _R, D), lambda i: (0, i, 0)),
            pl.BlockSpec((_R, D), lambda i: (i, 0)),
            pl.BlockSpec((_R, 1), lambda i: (i, 0)),
            pl.BlockSpec((1, D), lambda i: (0, 0)),
        ],
        out_specs=pl.BlockSpec((_R, D), lambda i: (i, 0)),
        out_shape=jax.ShapeDtypeStruct((N, D), jnp.float32),
    )(parts, h2p, dis, b2)


def kernel(x, edge_index, W1, b1, W2, b2):
    src = edge_index[0].astype(jnp.int32)
    dst = edge_index[1].astype(jnp.int32)

    pad_n = E_PAD - E
    pad_ar = jnp.arange(pad_n, dtype=jnp.int32)
    src_p = jnp.concatenate([src, pad_ar % N])
    dst_p = jnp.concatenate([dst, N + pad_ar % (N_PAD - N)])

    zvec = jnp.zeros((ROWS_PER_SC,), jnp.float32)
    zrows = jnp.zeros((80, D), jnp.float32)

    deg_parts = _deg_kernel(dst_p.reshape(-1, WIN), zvec)
    h1p, dis = _tc_b(x, W1, deg_parts)
    parts1 = _agg_kernel(h1p, src_p, dst_p, zrows)
    h2p = _tc_c(parts1, h1p, dis, W2, b1.reshape(1, D))
    parts2 = _agg_kernel(h2p, src_p, dst_p, zrows)
    return _tc_d(parts2, h2p, dis, b2.reshape(1, D))


# speedup vs baseline: 30.9707x; 1.1029x over previous
import jax
import jax.numpy as jnp
from jax import lax
from jax.experimental import pallas as pl
from jax.experimental.pallas import tpu as pltpu
from jax.experimental.pallas import tpu_sc as plsc

N = 10000
E = 320000
D = 128

N_PAD = 10240
NC = 2
NS = 16
WIN = 128
STEPS = 80
HALF = STEPS // 2
E_PAD = NC * NS * STEPS * WIN
NWIN = E_PAD // WIN
ROWS_PER_SC = N_PAD // NS

_MESH = plsc.VectorSubcoreMesh(core_axis_name="c", subcore_axis_name="s")


_PB = NWIN // 20


def _prep_body(e_ref, src_ref, dst_ref):
    k = pl.program_id(0)
    row = jax.lax.broadcasted_iota(jnp.int32, (_PB, WIN), 0)
    col = jax.lax.broadcasted_iota(jnp.int32, (_PB, WIN), 1)
    gidx = k * (_PB * WIN) + row * WIN + col
    real = gidx < E
    src = e_ref[0].reshape(_PB, WIN)
    dst = e_ref[1].reshape(_PB, WIN)
    src_ref[...] = jnp.where(real, src, gidx % N)
    dst_ref[...] = jnp.where(real, dst, N + gidx % (N_PAD - N))


@jax.jit
def _prep_kernel(edge_index):
    return pl.pallas_call(
        _prep_body,
        grid=(NWIN // _PB,),
        in_specs=[
            pl.BlockSpec((2, _PB * WIN), lambda k: (0, k)),
        ],
        out_specs=[
            pl.BlockSpec((_PB, WIN), lambda k: (k, 0)),
            pl.BlockSpec((_PB, WIN), lambda k: (k, 0)),
        ],
        out_shape=[
            jax.ShapeDtypeStruct((NWIN, WIN), jnp.int32),
            jax.ShapeDtypeStruct((NWIN, WIN), jnp.int32),
        ],
    )(edge_index)


def _deg_body(dst2d_hbm, zvec_hbm, out_hbm, dstv, onesv, sem, deg_sh):
    cid = lax.axis_index("c")
    sid = lax.axis_index("s")
    base = (cid * NS + sid) * STEPS

    pltpu.sync_copy(dst2d_hbm.at[pl.ds(base, STEPS)], dstv)
    for j in range(WIN // 16):
        onesv[pl.ds(j * 16, 16)] = jnp.ones((16,), jnp.float32)

    pltpu.sync_copy(zvec_hbm, deg_sh.at[pl.ds(sid * ROWS_PER_SC, ROWS_PER_SC)])
    plsc.subcore_barrier()

    def step(t, _):
        pltpu.sync_copy(onesv, deg_sh.at[dstv.at[t]], add=True)
        return 0

    lax.fori_loop(0, STEPS, step, 0)
    plsc.subcore_barrier()

    pltpu.sync_copy(
        deg_sh.at[pl.ds(sid * ROWS_PER_SC, ROWS_PER_SC)],
        out_hbm.at[cid, pl.ds(sid * ROWS_PER_SC, ROWS_PER_SC)],
    )


@jax.jit
def _deg_kernel(dst2d, zvec):
    return pl.kernel(
        _deg_body,
        out_type=jax.ShapeDtypeStruct((NC, N_PAD), jnp.float32),
        mesh=_MESH,
        scratch_types=[
            pltpu.VMEM((STEPS, WIN), jnp.int32),
            pltpu.VMEM((WIN,), jnp.float32),
            pltpu.SemaphoreType.DMA,
            pltpu.VMEM_SHARED((N_PAD,), jnp.float32),
        ],
    )(dst2d, zvec)


def _agg_body(h_hbm, src_hbm, dst_hbm, zrows_hbm, out_hbm,
              srcv, dstv, rows0, rows1, g0, g1, ss0, ss1, acc_sh):
    cid = lax.axis_index("c")
    sid = lax.axis_index("s")
    base = (cid * NS + sid) * STEPS
    rbufs = (rows0, rows1)
    gsems = (g0, g1)
    ssems = (ss0, ss1)

    pltpu.sync_copy(src_hbm.at[pl.ds(base, HALF)], srcv)
    pltpu.sync_copy(dst_hbm.at[pl.ds(base, HALF)], dstv)
    for k in range(ROWS_PER_SC // 80):
        pltpu.sync_copy(
            zrows_hbm, acc_sh.at[pl.ds(sid * ROWS_PER_SC + k * 80, 80)]
        )
    plsc.subcore_barrier()

    def run_half():
        pltpu.async_copy(h_hbm.at[srcv.at[0]], rbufs[0], gsems[0])

        def quad(i, _):
            for j in range(4):
                t = 4 * i + j
                b = j % 2
                nb = 1 - b

                @pl.when(t + 1 < HALF)
                def _():
                    pltpu.async_copy(
                        h_hbm.at[srcv.at[t + 1]], rbufs[nb], gsems[nb]
                    )

                pltpu.make_async_copy(
                    h_hbm.at[srcv.at[t]], rbufs[b], gsems[b]
                ).wait()
                pltpu.async_copy(
                    rbufs[b], acc_sh.at[dstv.at[t]], ssems[b], add=True
                )
                pltpu.make_async_copy(
                    rbufs[b], acc_sh.at[dstv.at[t]], ssems[b]
                ).wait()
            return 0

        lax.fori_loop(0, HALF // 4, quad, 0)

    run_half()
    pltpu.sync_copy(src_hbm.at[pl.ds(base + HALF, HALF)], srcv)
    pltpu.sync_copy(dst_hbm.at[pl.ds(base + HALF, HALF)], dstv)
    run_half()
    plsc.subcore_barrier()

    pltpu.sync_copy(
        acc_sh.at[pl.ds(sid * ROWS_PER_SC, ROWS_PER_SC)],
        out_hbm.at[cid, pl.ds(sid * ROWS_PER_SC, ROWS_PER_SC)],
    )


@jax.jit
def _agg_kernel(h, src2d, dst2d, zrows):
    return pl.kernel(
        _agg_body,
        out_type=jax.ShapeDtypeStruct((NC, N_PAD, D), jnp.float32),
        mesh=_MESH,
        scratch_types=(
            [pltpu.VMEM((HALF, WIN), jnp.int32)] * 2
            + [pltpu.VMEM((WIN, D), jnp.float32)] * 2
            + [pltpu.SemaphoreType.DMA] * 4
            + [pltpu.VMEM_SHARED((N_PAD, D), jnp.float32)]
        ),
    )(h, src2d, dst2d, zrows)


_R = 2048
_G = N_PAD // _R


def _tc_m1_body(x_ref, w_ref, h_ref):
    h_ref[...] = lax.dot_general(
        x_ref[...], w_ref[...], (((1,), (1,)), ((), ())),
        preferred_element_type=jnp.float32,
    )


@jax.jit
def _tc_m1(x, W1):
    return pl.pallas_call(
        _tc_m1_body,
        grid=(_G,),
        in_specs=[
            pl.BlockSpec((_R, D), lambda i: (i, 0)),
            pl.BlockSpec((D, D), lambda i: (0, 0)),
        ],
        out_specs=pl.BlockSpec((_R, D), lambda i: (i, 0)),
        out_shape=jax.ShapeDtypeStruct((N_PAD, D), jnp.float32),
    )(x, W1)


def _tc_scale_body(h_ref, deg_ref, hs_ref, dis_ref):
    deg = deg_ref[0, :] + deg_ref[1, :] + 1.0
    dis = lax.rsqrt(deg)[:, None]
    hs_ref[...] = h_ref[...] * dis
    dis_ref[...] = dis


@jax.jit
def _tc_scale(hlin, deg_parts):
    return pl.pallas_call(
        _tc_scale_body,
        grid=(_G,),
        in_specs=[
            pl.BlockSpec((_R, D), lambda i: (i, 0)),
            pl.BlockSpec((NC, _R), lambda i: (0, i)),
        ],
        out_specs=[
            pl.BlockSpec((_R, D), lambda i: (i, 0)),
            pl.BlockSpec((_R, 1), lambda i: (i, 0)),
        ],
        out_shape=[
            jax.ShapeDtypeStruct((N_PAD, D), jnp.float32),
            jax.ShapeDtypeStruct((N_PAD, 1), jnp.float32),
        ],
    )(hlin, deg_parts)


def _tc_c_body(parts_ref, h_ref, dis_ref, w_ref, b_ref, out_ref):
    dis = dis_ref[...]
    z = dis * (parts_ref[0] + parts_ref[1] + h_ref[...]) + b_ref[...]
    h = jnp.maximum(z, 0.0)
    out_ref[...] = dis * lax.dot_general(
        h, w_ref[...], (((1,), (1,)), ((), ())),
        preferred_element_type=jnp.float32,
    )


@jax.jit
def _tc_c(parts, h1p, dis, W2, b1):
    return pl.pallas_call(
        _tc_c_body,
        grid=(_G,),
        in_specs=[
            pl.BlockSpec((NC, _R, D), lambda i: (0, i, 0)),
            pl.BlockSpec((_R, D), lambda i: (i, 0)),
            pl.BlockSpec((_R, 1), lambda i: (i, 0)),
            pl.BlockSpec((D, D), lambda i: (0, 0)),
            pl.BlockSpec((1, D), lambda i: (0, 0)),
        ],
        out_specs=pl.BlockSpec((_R, D), lambda i: (i, 0)),
        out_shape=jax.ShapeDtypeStruct((N_PAD, D), jnp.float32),
    )(parts, h1p, dis, W2, b1)


def _tc_d_body(parts_ref, h_ref, dis_ref, b_ref, out_ref):
    out_ref[...] = (
        dis_ref[...] * (parts_ref[0] + parts_ref[1] + h_ref[...]) + b_ref[...]
    )


@jax.jit
def _tc_d(parts, h2p, dis, b2):
    return pl.pallas_call(
        _tc_d_body,
        grid=(_G,),
        in_specs=[
            pl.BlockSpec((NC, _R, D), lambda i: (0, i, 0)),
            pl.BlockSpec((_R, D), lambda i: (i, 0)),
            pl.BlockSpec((_R, 1), lambda i: (i, 0)),
            pl.BlockSpec((1, D), lambda i: (0, 0)),
        ],
        out_specs=pl.BlockSpec((_R, D), lambda i: (i, 0)),
        out_shape=jax.ShapeDtypeStruct((N, D), jnp.float32),
    )(parts, h2p, dis, b2)


def kernel(x, edge_index, W1, b1, W2, b2):
    src2d, dst2d = _prep_kernel(edge_index.astype(jnp.int32))
    zvec = jnp.zeros((ROWS_PER_SC,), jnp.float32)
    zrows = jnp.zeros((80, D), jnp.float32)

    deg_parts = _deg_kernel(dst2d, zvec)
    hlin = _tc_m1(x, W1)
    h1p, dis = _tc_scale(hlin, deg_parts)
    parts1 = _agg_kernel(h1p, src2d, dst2d, zrows)
    h2p = _tc_c(parts1, h1p, dis, W2, b1.reshape(1, D))
    parts2 = _agg_kernel(h2p, src2d, dst2d, zrows)
    return _tc_d(parts2, h2p, dis, b2.reshape(1, D))
